# Initial kernel scaffold; baseline (speedup 1.0000x reference)
import jax
import jax.numpy as jnp
from jax.experimental import pallas as pl


def kernel(x, emb_weight, na_param):
    raise NotImplementedError("write your pallas kernel here")


# speedup vs baseline: 6.1024x; 6.1024x over previous
import jax
import jax.numpy as jnp
from jax import lax
from jax.experimental import pallas as pl
from jax.experimental.pallas import tpu as pltpu
from jax.experimental.pallas import tpu_sc as plsc

N = 16777216
NC = 2
NS = 16
NW = NC * NS
PER_W = N // NW
L = 16
CHUNK = 8192
NCHUNK = PER_W // CHUNK
VPC = CHUNK // L


def _body(x_hbm, emb_hbm, na_hbm, out_hbm, emb_v, na_v, in_v, out_v):
    wid = lax.axis_index("s") * NC + lax.axis_index("c")
    base = wid * PER_W
    pltpu.sync_copy(emb_hbm, emb_v)
    pltpu.sync_copy(na_hbm, na_v)
    na_vec = na_v[...]
    tab_lo = emb_v[pl.ds(0, L)]
    tab_hi = emb_v[pl.ds(L, L)]

    def chunk_body(c, carry):
        start = base + c * CHUNK
        pltpu.sync_copy(x_hbm.at[pl.ds(start, CHUNK)], in_v)

        def vec_body(i, carry2):
            v = in_v[pl.ds(i * L, L)]
            t = v * 4.0 + 12.0
            ii = t.astype(jnp.int32)
            ii = ii + jnp.where(ii.astype(jnp.float32) < t, 1, 0) + 1
            ii = jnp.minimum(jnp.maximum(ii, 1), 26)
            y_lo = tab_lo.at[jnp.minimum(ii, L - 1)].get(
                mode="promise_in_bounds")
            y_hi = tab_hi.at[jnp.maximum(ii - L, 0)].get(
                mode="promise_in_bounds")
            y = jnp.where(ii < L, y_lo, y_hi)
            out_v[pl.ds(i * L, L)] = jnp.where(v + 999.0 < 1e-6, na_vec, y)
            return carry2

        lax.fori_loop(0, VPC, vec_body, 0)
        pltpu.sync_copy(out_v, out_hbm.at[pl.ds(start, CHUNK)])
        return carry

    lax.fori_loop(0, NCHUNK, chunk_body, 0)


def kernel(x, emb_weight, na_param):
    emb_pad = jnp.pad(emb_weight.astype(jnp.float32), (0, 32 - emb_weight.shape[0]))
    na_vec = jnp.full((L,), na_param[0], dtype=jnp.float32)
    k = pl.kernel(
        _body,
        out_type=jax.ShapeDtypeStruct((N,), jnp.float32),
        mesh=plsc.VectorSubcoreMesh(core_axis_name="c", subcore_axis_name="s"),
        scratch_types=[
            pltpu.VMEM((32,), jnp.float32),
            pltpu.VMEM((L,), jnp.float32),
            pltpu.VMEM((CHUNK,), jnp.float32),
            pltpu.VMEM((CHUNK,), jnp.float32),
        ],
    )
    out = k(x.astype(jnp.float32), emb_pad, na_vec)
    return out.reshape(1, N)


# speedup vs baseline: 9.3352x; 1.5298x over previous
import jax
import jax.numpy as jnp
from jax import lax
from jax.experimental import pallas as pl
from jax.experimental.pallas import tpu as pltpu
from jax.experimental.pallas import tpu_sc as plsc

N = 16777216
NC = 2
NS = 16
NW = NC * NS
PER_W = N // NW
L = 16
CHUNK = 16384
NCHUNK = PER_W // CHUNK
NGRP = NCHUNK // 2
VPC = CHUNK // L
U = 8


def _body(x_hbm, emb_hbm, na_hbm, out_hbm,
          emb_v, na_v, in0, in1, ob0, ob1,
          isem0, isem1, osem0, osem1):
    wid = lax.axis_index("s") * NC + lax.axis_index("c")
    base = wid * PER_W
    pltpu.sync_copy(emb_hbm, emb_v)
    pltpu.sync_copy(na_hbm, na_v)
    na_vec = na_v[...]
    tab_lo = emb_v[pl.ds(0, L)]
    tab_hi = emb_v[pl.ds(L, L)]

    def in_copy(c, buf, sem):
        return pltpu.make_async_copy(
            x_hbm.at[pl.ds(base + c * CHUNK, CHUNK)], buf, sem)

    def out_copy(c, buf, sem):
        return pltpu.make_async_copy(
            buf, out_hbm.at[pl.ds(base + c * CHUNK, CHUNK)], sem)

    def compute(src, dst):
        def step(i, carry):
            b = i * (U * L)
            for u in range(U):
                v = src[pl.ds(b + u * L, L)]
                t = v * 4.0 + 12.0
                ii = t.astype(jnp.int32)
                ii = ii + jnp.where(ii.astype(jnp.float32) < t, 1, 0) + 1
                y_lo = tab_lo.at[jnp.minimum(jnp.maximum(ii, 1), L - 1)].get(
                    mode="promise_in_bounds")
                y_hi = tab_hi.at[jnp.maximum(jnp.minimum(ii, 26) - L, 0)].get(
                    mode="promise_in_bounds")
                y = jnp.where(ii < L, y_lo, y_hi)
                dst[pl.ds(b + u * L, L)] = jnp.where(
                    v + 999.0 < 1e-6, na_vec, y)
            return carry
        lax.fori_loop(0, VPC // U, step, 0)

    in_copy(0, in0, isem0).start()
    in_copy(1, in1, isem1).start()

    def group(g, carry):
        ca = 2 * g
        in_copy(ca, in0, isem0).wait()

        @pl.when(g > 0)
        def _():
            out_copy(ca - 2, ob0, osem0).wait()
        compute(in0, ob0)
        out_copy(ca, ob0, osem0).start()

        @pl.when(g < NGRP - 1)
        def _():
            in_copy(ca + 2, in0, isem0).start()

        in_copy(ca + 1, in1, isem1).wait()

        @pl.when(g > 0)
        def _():
            out_copy(ca - 1, ob1, osem1).wait()
        compute(in1, ob1)
        out_copy(ca + 1, ob1, osem1).start()

        @pl.when(g < NGRP - 1)
        def _():
            in_copy(ca + 3, in1, isem1).start()
        return carry

    lax.fori_loop(0, NGRP, group, 0)
    out_copy(NCHUNK - 2, ob0, osem0).wait()
    out_copy(NCHUNK - 1, ob1, osem1).wait()


def kernel(x, emb_weight, na_param):
    emb_pad = jnp.pad(emb_weight.astype(jnp.float32), (0, 32 - emb_weight.shape[0]))
    na_vec = jnp.full((L,), na_param[0], dtype=jnp.float32)
    k = pl.kernel(
        _body,
        out_type=jax.ShapeDtypeStruct((N,), jnp.float32),
        mesh=plsc.VectorSubcoreMesh(core_axis_name="c", subcore_axis_name="s"),
        scratch_types=[
            pltpu.VMEM((32,), jnp.float32),
            pltpu.VMEM((L,), jnp.float32),
            pltpu.VMEM((CHUNK,), jnp.float32),
            pltpu.VMEM((CHUNK,), jnp.float32),
            pltpu.VMEM((CHUNK,), jnp.float32),
            pltpu.VMEM((CHUNK,), jnp.float32),
            pltpu.SemaphoreType.DMA,
            pltpu.SemaphoreType.DMA,
            pltpu.SemaphoreType.DMA,
            pltpu.SemaphoreType.DMA,
        ],
    )
    out = k(x.astype(jnp.float32), emb_pad, na_vec)
    return out.reshape(1, N)


# speedup vs baseline: 10.6332x; 1.1390x over previous
import jax
import jax.numpy as jnp
from jax import lax
from jax.experimental import pallas as pl
from jax.experimental.pallas import tpu as pltpu
from jax.experimental.pallas import tpu_sc as plsc

N = 16777216
NC = 2
NS = 16
NW = NC * NS
PER_W = N // NW
L = 16
CHUNK = 16384
NCHUNK = PER_W // CHUNK
NGRP = NCHUNK // 2
VPC = CHUNK // L
U = 8


def _body(x_hbm, emb_hbm, na_hbm, out_hbm,
          emb_v, na_v, in0, in1, ob0, ob1,
          isem0, isem1, osem0, osem1):
    wid = lax.axis_index("s") * NC + lax.axis_index("c")
    base = wid * PER_W
    pltpu.sync_copy(emb_hbm, emb_v)
    pltpu.sync_copy(na_hbm, na_v)
    na_vec = na_v[...]
    tab_lo = emb_v[pl.ds(0, L)]
    tab_hi = emb_v[pl.ds(L, L)]

    def in_copy(c, buf, sem):
        return pltpu.make_async_copy(
            x_hbm.at[pl.ds(base + c * CHUNK, CHUNK)], buf, sem)

    def out_copy(c, buf, sem):
        return pltpu.make_async_copy(
            buf, out_hbm.at[pl.ds(base + c * CHUNK, CHUNK)], sem)

    def compute(src, dst):
        def step(i, carry):
            b = i * (U * L)
            for u in range(U):
                v = src[pl.ds(b + u * L, L)]
                ii = (v * 4.0 + 12.999984741210938).astype(jnp.int32)
                y_lo = tab_lo.at[jnp.minimum(jnp.maximum(ii, 0), L - 1)].get(
                    mode="promise_in_bounds")
                y_hi = tab_hi.at[jnp.maximum(jnp.minimum(ii, 25) - L, 0)].get(
                    mode="promise_in_bounds")
                y = jnp.where(ii < L, y_lo, y_hi)
                dst[pl.ds(b + u * L, L)] = jnp.where(v <= -999.0, na_vec, y)
            return carry
        lax.fori_loop(0, VPC // U, step, 0)

    in_copy(0, in0, isem0).start()
    in_copy(1, in1, isem1).start()

    def group(g, carry):
        ca = 2 * g
        in_copy(ca, in0, isem0).wait()

        @pl.when(g > 0)
        def _():
            out_copy(ca - 2, ob0, osem0).wait()
        compute(in0, ob0)
        out_copy(ca, ob0, osem0).start()

        @pl.when(g < NGRP - 1)
        def _():
            in_copy(ca + 2, in0, isem0).start()

        in_copy(ca + 1, in1, isem1).wait()

        @pl.when(g > 0)
        def _():
            out_copy(ca - 1, ob1, osem1).wait()
        compute(in1, ob1)
        out_copy(ca + 1, ob1, osem1).start()

        @pl.when(g < NGRP - 1)
        def _():
            in_copy(ca + 3, in1, isem1).start()
        return carry

    lax.fori_loop(0, NGRP, group, 0)
    out_copy(NCHUNK - 2, ob0, osem0).wait()
    out_copy(NCHUNK - 1, ob1, osem1).wait()


def kernel(x, emb_weight, na_param):
    emb_pad = jnp.pad(emb_weight.astype(jnp.float32)[1:],
                      (0, 33 - emb_weight.shape[0]))
    na_vec = jnp.full((L,), na_param[0], dtype=jnp.float32)
    k = pl.kernel(
        _body,
        out_type=jax.ShapeDtypeStruct((N,), jnp.float32),
        mesh=plsc.VectorSubcoreMesh(core_axis_name="c", subcore_axis_name="s"),
        scratch_types=[
            pltpu.VMEM((32,), jnp.float32),
            pltpu.VMEM((L,), jnp.float32),
            pltpu.VMEM((CHUNK,), jnp.float32),
            pltpu.VMEM((CHUNK,), jnp.float32),
            pltpu.VMEM((CHUNK,), jnp.float32),
            pltpu.VMEM((CHUNK,), jnp.float32),
            pltpu.SemaphoreType.DMA,
            pltpu.SemaphoreType.DMA,
            pltpu.SemaphoreType.DMA,
            pltpu.SemaphoreType.DMA,
        ],
    )
    out = k(x.astype(jnp.float32), emb_pad, na_vec)
    return out.reshape(1, N)


# speedup vs baseline: 11.1483x; 1.0484x over previous
import jax
import jax.numpy as jnp
from jax import lax
from jax.experimental import pallas as pl
from jax.experimental.pallas import tpu as pltpu
from jax.experimental.pallas import tpu_sc as plsc

N = 16777216
NC = 2
NS = 16
NW = NC * NS
PER_W = N // NW
L = 16
CHUNK = 16384
NCHUNK = PER_W // CHUNK
NGRP = NCHUNK // 2
VPC = CHUNK // L
U = 8


def _body(x_hbm, emb_hbm, na_hbm, out_hbm,
          emb_v, na_v, in0, in1, ob0, ob1,
          isem0, isem1, osem0, osem1):
    wid = lax.axis_index("s") * NC + lax.axis_index("c")
    base = wid * PER_W
    pltpu.sync_copy(emb_hbm, emb_v)
    pltpu.sync_copy(na_hbm, na_v)
    na_vec = na_v[...]
    tab_lo = emb_v[pl.ds(0, L)]
    tab_hi = emb_v[pl.ds(L, L)]

    def in_copy(c, buf, sem):
        return pltpu.make_async_copy(
            x_hbm.at[pl.ds(base + c * CHUNK, CHUNK)], buf, sem)

    def out_copy(c, buf, sem):
        return pltpu.make_async_copy(
            buf, out_hbm.at[pl.ds(base + c * CHUNK, CHUNK)], sem)

    def compute(src, dst):
        @plsc.parallel_loop(0, CHUNK, step=L, unroll=U)
        def _loop(i):
            v = src[pl.ds(i, L)]
            ii = (v * 4.0 + 12.999984741210938).astype(jnp.int32)
            y_lo = tab_lo.at[jnp.minimum(jnp.maximum(ii, 0), L - 1)].get(
                mode="promise_in_bounds")
            y_hi = tab_hi.at[jnp.maximum(jnp.minimum(ii, 25) - L, 0)].get(
                mode="promise_in_bounds")
            y = jnp.where(ii < L, y_lo, y_hi)
            dst[pl.ds(i, L)] = jnp.where(v <= -999.0, na_vec, y)

    in_copy(0, in0, isem0).start()
    in_copy(1, in1, isem1).start()

    def group(g, carry):
        ca = 2 * g
        in_copy(ca, in0, isem0).wait()

        @pl.when(g > 0)
        def _():
            out_copy(ca - 2, ob0, osem0).wait()
        compute(in0, ob0)
        out_copy(ca, ob0, osem0).start()

        @pl.when(g < NGRP - 1)
        def _():
            in_copy(ca + 2, in0, isem0).start()

        in_copy(ca + 1, in1, isem1).wait()

        @pl.when(g > 0)
        def _():
            out_copy(ca - 1, ob1, osem1).wait()
        compute(in1, ob1)
        out_copy(ca + 1, ob1, osem1).start()

        @pl.when(g < NGRP - 1)
        def _():
            in_copy(ca + 3, in1, isem1).start()
        return carry

    lax.fori_loop(0, NGRP, group, 0)
    out_copy(NCHUNK - 2, ob0, osem0).wait()
    out_copy(NCHUNK - 1, ob1, osem1).wait()


def kernel(x, emb_weight, na_param):
    emb_pad = jnp.pad(emb_weight.astype(jnp.float32)[1:],
                      (0, 33 - emb_weight.shape[0]))
    na_vec = jnp.full((L,), na_param[0], dtype=jnp.float32)
    k = pl.kernel(
        _body,
        out_type=jax.ShapeDtypeStruct((N,), jnp.float32),
        mesh=plsc.VectorSubcoreMesh(core_axis_name="c", subcore_axis_name="s"),
        scratch_types=[
            pltpu.VMEM((32,), jnp.float32),
            pltpu.VMEM((L,), jnp.float32),
            pltpu.VMEM((CHUNK,), jnp.float32),
            pltpu.VMEM((CHUNK,), jnp.float32),
            pltpu.VMEM((CHUNK,), jnp.float32),
            pltpu.VMEM((CHUNK,), jnp.float32),
            pltpu.SemaphoreType.DMA,
            pltpu.SemaphoreType.DMA,
            pltpu.SemaphoreType.DMA,
            pltpu.SemaphoreType.DMA,
        ],
    )
    out = k(x.astype(jnp.float32), emb_pad, na_vec)
    return out.reshape(1, N)


# speedup vs baseline: 12.1887x; 1.0933x over previous
import jax
import jax.numpy as jnp
from jax import lax
from jax.experimental import pallas as pl
from jax.experimental.pallas import tpu as pltpu
from jax.experimental.pallas import tpu_sc as plsc

N = 16777216
NC = 2
NS = 16
NW = NC * NS
PER_W = N // NW
L = 16
CHUNK = 16384
NCHUNK = PER_W // CHUNK
NGRP = NCHUNK // 2
VPC = CHUNK // L
U = 8


def _body(x_hbm, emb_hbm, na_hbm, out_hbm,
          emb_v, na_v, in0, in1, ob0, ob1,
          isem0, isem1, osem0, osem1):
    wid = lax.axis_index("s") * NC + lax.axis_index("c")
    base = wid * PER_W
    pltpu.sync_copy(emb_hbm, emb_v)
    pltpu.sync_copy(na_hbm, na_v)
    na_vec = na_v[...]
    tab_lo = emb_v[pl.ds(0, L)]
    tab_hi = emb_v[pl.ds(L, L)]

    def in_copy(c, buf, sem):
        return pltpu.make_async_copy(
            x_hbm.at[pl.ds(base + c * CHUNK, CHUNK)], buf, sem)

    def out_copy(c, buf, sem):
        return pltpu.make_async_copy(
            buf, out_hbm.at[pl.ds(base + c * CHUNK, CHUNK)], sem)

    def compute(src, dst):
        @plsc.parallel_loop(0, CHUNK, step=L, unroll=U)
        def _loop(i):
            v = src[pl.ds(i, L)]
            u_f = v * 4.0 + 12.999984741210938
            ii = jnp.minimum(jnp.maximum(u_f, 0.0), 25.5).astype(jnp.int32)
            y_lo = tab_lo.at[jnp.minimum(ii, L - 1)].get(
                mode="promise_in_bounds")
            y_hi = tab_hi.at[jnp.maximum(ii - (L - 1), 0)].get(
                mode="promise_in_bounds")
            dst[pl.ds(i, L)] = jnp.where(v <= -999.0, na_vec, y_lo + y_hi)

    in_copy(0, in0, isem0).start()
    in_copy(1, in1, isem1).start()

    def group(g, carry):
        ca = 2 * g
        in_copy(ca, in0, isem0).wait()

        @pl.when(g > 0)
        def _():
            out_copy(ca - 2, ob0, osem0).wait()
        compute(in0, ob0)
        out_copy(ca, ob0, osem0).start()

        @pl.when(g < NGRP - 1)
        def _():
            in_copy(ca + 2, in0, isem0).start()

        in_copy(ca + 1, in1, isem1).wait()

        @pl.when(g > 0)
        def _():
            out_copy(ca - 1, ob1, osem1).wait()
        compute(in1, ob1)
        out_copy(ca + 1, ob1, osem1).start()

        @pl.when(g < NGRP - 1)
        def _():
            in_copy(ca + 3, in1, isem1).start()
        return carry

    lax.fori_loop(0, NGRP, group, 0)
    out_copy(NCHUNK - 2, ob0, osem0).wait()
    out_copy(NCHUNK - 1, ob1, osem1).wait()


def kernel(x, emb_weight, na_param):
    ew = emb_weight.astype(jnp.float32)
    tab_a = ew[1:17]
    tab_b = jnp.pad(ew[17:27] - ew[16], (1, 5))
    emb_pad = jnp.concatenate([tab_a, tab_b])
    na_vec = jnp.full((L,), na_param[0], dtype=jnp.float32)
    k = pl.kernel(
        _body,
        out_type=jax.ShapeDtypeStruct((N,), jnp.float32),
        mesh=plsc.VectorSubcoreMesh(core_axis_name="c", subcore_axis_name="s"),
        scratch_types=[
            pltpu.VMEM((32,), jnp.float32),
            pltpu.VMEM((L,), jnp.float32),
            pltpu.VMEM((CHUNK,), jnp.float32),
            pltpu.VMEM((CHUNK,), jnp.float32),
            pltpu.VMEM((CHUNK,), jnp.float32),
            pltpu.VMEM((CHUNK,), jnp.float32),
            pltpu.SemaphoreType.DMA,
            pltpu.SemaphoreType.DMA,
            pltpu.SemaphoreType.DMA,
            pltpu.SemaphoreType.DMA,
        ],
    )
    out = k(x.astype(jnp.float32), emb_pad, na_vec)
    return out.reshape(1, N)
